# Initial kernel scaffold; baseline (speedup 1.0000x reference)
import jax
import jax.numpy as jnp
from jax.experimental import pallas as pl


def kernel(x, Wqk, bqk, Wv, bv, Wo, bo, rotations):
    raise NotImplementedError("write your pallas kernel here")


# speedup vs baseline: 1.1508x; 1.1508x over previous
import functools
import math

import jax
import jax.numpy as jnp
from jax.experimental import pallas as pl
from jax.experimental.pallas import tpu as pltpu


def _proj_kernel(x_ref, wqk_ref, bqk_ref, wv_ref, bv_ref, qk_ref, v_ref):
    x = x_ref[...]
    qk_ref[...] = jax.lax.dot_general(
        x, wqk_ref[...], (((1,), (1,)), ((), ())),
        preferred_element_type=jnp.float32) + bqk_ref[...]
    v_ref[...] = jax.lax.dot_general(
        x, wv_ref[...], (((1,), (1,)), ((), ())),
        preferred_element_type=jnp.float32) + bv_ref[...]


def _project(x2, Wqk, bqk, Wv, bv, row_block):
    S, DM = x2.shape
    grid = (S // row_block,)
    return pl.pallas_call(
        _proj_kernel,
        grid=grid,
        in_specs=[
            pl.BlockSpec((row_block, DM), lambda i: (i, 0)),
            pl.BlockSpec((DM, DM), lambda i: (0, 0)),
            pl.BlockSpec((1, DM), lambda i: (0, 0)),
            pl.BlockSpec((DM, DM), lambda i: (0, 0)),
            pl.BlockSpec((1, DM), lambda i: (0, 0)),
        ],
        out_specs=[
            pl.BlockSpec((row_block, DM), lambda i: (i, 0)),
            pl.BlockSpec((row_block, DM), lambda i: (i, 0)),
        ],
        out_shape=[
            jax.ShapeDtypeStruct((S, DM), jnp.float32),
            jax.ShapeDtypeStruct((S, DM), jnp.float32),
        ],
    )(x2, Wqk, bqk, Wv, bv)


def _hash_kernel(qk_ref, rot_ref, bkt_ref, *, n_hashes, rot_size):
    q = qk_ref[...]
    r = rot_ref[0]
    rot = jax.lax.dot_general(
        q, r, (((1,), (0,)), ((), ())), preferred_element_type=jnp.float32)
    for h in range(n_hashes):
        sub = rot[:, h * rot_size:(h + 1) * rot_size]
        full = jnp.concatenate([sub, -sub], axis=1)
        bkt_ref[0, h, :] = jnp.argmax(full, axis=1).astype(jnp.int32)


def _hash_buckets(qk_heads, rot_flat, n_hashes, rot_size):
    H = rot_flat.shape[0]
    D = rot_flat.shape[1]
    S = qk_heads.shape[0] // H
    return pl.pallas_call(
        functools.partial(_hash_kernel, n_hashes=n_hashes, rot_size=rot_size),
        grid=(H,),
        in_specs=[
            pl.BlockSpec((S, D), lambda h: (h, 0)),
            pl.BlockSpec((1, D, n_hashes * rot_size), lambda h: (h, 0, 0)),
        ],
        out_specs=pl.BlockSpec((1, n_hashes, S), lambda h: (h, 0, 0)),
        out_shape=jax.ShapeDtypeStruct((H, n_hashes, S), jnp.int32),
    )(qk_heads, rot_flat)


def _attn_kernel(bq_ref, bk_ref, q_ref, k_ref, v_ref, o_ref,
                 m_ref, l_ref, acc_ref, acct_ref,
                 *, n_hashes, nk, scale):
    h = pl.program_id(1)
    ki = pl.program_id(2)

    @pl.when(jnp.logical_and(h == 0, ki == 0))
    def _():
        acct_ref[...] = jnp.zeros_like(acct_ref)

    @pl.when(ki == 0)
    def _():
        m_ref[...] = jnp.full_like(m_ref, -1e30)
        l_ref[...] = jnp.zeros_like(l_ref)
        acc_ref[...] = jnp.zeros_like(acc_ref)

    q = q_ref[...]
    k = k_ref[...]
    s = jax.lax.dot_general(
        q, k, (((1,), (1,)), ((), ())),
        preferred_element_type=jnp.float32) * scale
    bq = bq_ref[0, 0, :]
    bk = bk_ref[0, 0, :]
    mask = bq[:, None] == bk[None, :]
    s = jnp.where(mask, s, -1e30)

    m_prev = m_ref[...]
    m_new = jnp.maximum(m_prev, jnp.max(s, axis=1, keepdims=True))
    alpha = jnp.exp(m_prev - m_new)
    p = jnp.exp(s - m_new)
    l_ref[...] = l_ref[...] * alpha + jnp.sum(p, axis=1, keepdims=True)
    acc_ref[...] = acc_ref[...] * alpha + jax.lax.dot_general(
        p, v_ref[...], (((1,), (0,)), ((), ())),
        preferred_element_type=jnp.float32)
    m_ref[...] = m_new

    @pl.when(ki == nk - 1)
    def _():
        acct_ref[...] = acct_ref[...] + acc_ref[...] / l_ref[...]

    @pl.when(jnp.logical_and(h == n_hashes - 1, ki == nk - 1))
    def _():
        o_ref[...] = acct_ref[...] * (1.0 / n_hashes)


def _attention(qk_flat, v_flat, buckets3, bq_block, bk_block):
    T, D = qk_flat.shape
    n_hashes = buckets3.shape[0]
    nq = T // bq_block
    nk = T // bk_block
    scale = 1.0 / math.sqrt(D)
    return pl.pallas_call(
        functools.partial(_attn_kernel, n_hashes=n_hashes, nk=nk, scale=scale),
        grid=(nq, n_hashes, nk),
        in_specs=[
            pl.BlockSpec((1, 1, bq_block), lambda qi, h, ki: (h, 0, qi)),
            pl.BlockSpec((1, 1, bk_block), lambda qi, h, ki: (h, 0, ki)),
            pl.BlockSpec((bq_block, D), lambda qi, h, ki: (qi, 0)),
            pl.BlockSpec((bk_block, D), lambda qi, h, ki: (ki, 0)),
            pl.BlockSpec((bk_block, D), lambda qi, h, ki: (ki, 0)),
        ],
        out_specs=pl.BlockSpec((bq_block, D), lambda qi, h, ki: (qi, 0)),
        out_shape=jax.ShapeDtypeStruct((T, D), jnp.float32),
        scratch_shapes=[
            pltpu.VMEM((bq_block, 1), jnp.float32),
            pltpu.VMEM((bq_block, 1), jnp.float32),
            pltpu.VMEM((bq_block, D), jnp.float32),
            pltpu.VMEM((bq_block, D), jnp.float32),
        ],
        compiler_params=pltpu.CompilerParams(
            dimension_semantics=("parallel", "arbitrary", "arbitrary")),
    )(buckets3, buckets3, qk_flat, qk_flat, v_flat)


def _outproj_kernel(y_ref, wo_ref, bo_ref, o_ref):
    o_ref[...] = jax.lax.dot_general(
        y_ref[...], wo_ref[...], (((1,), (1,)), ((), ())),
        preferred_element_type=jnp.float32) + bo_ref[...]


def _outproj(y2, Wo, bo, row_block):
    S, DM = y2.shape
    return pl.pallas_call(
        _outproj_kernel,
        grid=(S // row_block,),
        in_specs=[
            pl.BlockSpec((row_block, DM), lambda i: (i, 0)),
            pl.BlockSpec((DM, DM), lambda i: (0, 0)),
            pl.BlockSpec((1, DM), lambda i: (0, 0)),
        ],
        out_specs=pl.BlockSpec((row_block, DM), lambda i: (i, 0)),
        out_shape=jax.ShapeDtypeStruct((S, DM), jnp.float32),
    )(y2, Wo, bo)


def kernel(x, Wqk, bqk, Wv, bv, Wo, bo, rotations):
    batch, S, DM = x.shape
    n_hashes, H, D, C = rotations.shape
    T = batch * H * S

    x2 = x.reshape(batch * S, DM)
    row_block = min(256, batch * S)
    qk2, v2 = _project(x2, Wqk, bqk.reshape(1, DM), Wv, bv.reshape(1, DM),
                       row_block)

    qk_flat = qk2.reshape(batch, S, H, D).transpose(0, 2, 1, 3).reshape(T, D)
    v_flat = v2.reshape(batch, S, H, D).transpose(0, 2, 1, 3).reshape(T, D)

    rot_flat = rotations.transpose(1, 2, 0, 3).reshape(H, D, n_hashes * C)
    bkt = _hash_buckets(qk_flat, rot_flat, n_hashes, C)
    buckets = bkt.transpose(1, 0, 2).reshape(n_hashes, 1, T)

    bq_block = min(512, T)
    bk_block = min(1024, T)
    out_flat = _attention(qk_flat, v_flat, buckets, bq_block, bk_block)

    y2 = out_flat.reshape(batch, H, S, D).transpose(0, 2, 1, 3).reshape(
        batch * S, DM)
    out = _outproj(y2, Wo, bo.reshape(1, DM), row_block)
    return out.reshape(batch, S, DM)


# speedup vs baseline: 4.6919x; 4.0770x over previous
import functools
import math

import jax
import jax.numpy as jnp
from jax import lax
from jax.experimental import pallas as pl
from jax.experimental.pallas import tpu as pltpu
from jax.experimental.pallas import tpu_sc as plsc


def _proj_kernel(x_ref, wqk_ref, bqk_ref, wv_ref, bv_ref, qk_ref, v_ref):
    x = x_ref[...]
    qk_ref[...] = jax.lax.dot_general(
        x, wqk_ref[...], (((1,), (1,)), ((), ())),
        preferred_element_type=jnp.float32) + bqk_ref[...]
    v_ref[...] = jax.lax.dot_general(
        x, wv_ref[...], (((1,), (1,)), ((), ())),
        preferred_element_type=jnp.float32) + bv_ref[...]


def _project(x2, Wqk, bqk, Wv, bv, row_block):
    S, DM = x2.shape
    return pl.pallas_call(
        _proj_kernel,
        grid=(S // row_block,),
        in_specs=[
            pl.BlockSpec((row_block, DM), lambda i: (i, 0)),
            pl.BlockSpec((DM, DM), lambda i: (0, 0)),
            pl.BlockSpec((1, DM), lambda i: (0, 0)),
            pl.BlockSpec((DM, DM), lambda i: (0, 0)),
            pl.BlockSpec((1, DM), lambda i: (0, 0)),
        ],
        out_specs=[
            pl.BlockSpec((row_block, DM), lambda i: (i, 0)),
            pl.BlockSpec((row_block, DM), lambda i: (i, 0)),
        ],
        out_shape=[
            jax.ShapeDtypeStruct((S, DM), jnp.float32),
            jax.ShapeDtypeStruct((S, DM), jnp.float32),
        ],
    )(x2, Wqk, bqk, Wv, bv)


def _hash_kernel(qk_ref, rot_ref, bkt_ref, *, n_hashes, rot_size):
    q = qk_ref[...]
    r = rot_ref[0]
    rot = jax.lax.dot_general(
        q, r, (((1,), (0,)), ((), ())), preferred_element_type=jnp.float32)
    for h in range(n_hashes):
        sub = rot[:, h * rot_size:(h + 1) * rot_size]
        full = jnp.concatenate([sub, -sub], axis=1)
        bkt_ref[0, h, :] = jnp.argmax(full, axis=1).astype(jnp.int32)


def _hash_buckets(qk_heads, rot_flat, n_hashes, rot_size):
    H = rot_flat.shape[0]
    D = rot_flat.shape[1]
    S = qk_heads.shape[0] // H
    return pl.pallas_call(
        functools.partial(_hash_kernel, n_hashes=n_hashes, rot_size=rot_size),
        grid=(H,),
        in_specs=[
            pl.BlockSpec((S, D), lambda h: (h, 0)),
            pl.BlockSpec((1, D, n_hashes * rot_size), lambda h: (h, 0, 0)),
        ],
        out_specs=pl.BlockSpec((1, n_hashes, S), lambda h: (h, 0, 0)),
        out_shape=jax.ShapeDtypeStruct((H, n_hashes, S), jnp.int32),
    )(qk_heads, rot_flat)


def _rank_kernel(bkt_ref, rank_ref, offs_ref, counts_sc, offs_sc,
                 *, cs, nc, nb, t):
    h = pl.program_id(0)
    ph = pl.program_id(1)
    c = pl.program_id(2)

    b = bkt_ref[0, 0, :]
    lanes = lax.broadcasted_iota(jnp.int32, (cs, nb), 1)
    oh = (b[:, None] == lanes).astype(jnp.float32)

    @pl.when(jnp.logical_and(ph == 0, c == 0))
    def _():
        counts_sc[...] = jnp.zeros_like(counts_sc)

    @pl.when(ph == 0)
    def _():
        counts_sc[...] = counts_sc[...] + jnp.sum(oh, axis=0, keepdims=True)

    @pl.when(jnp.logical_and(ph == 0, c == nc - 1))
    def _():
        cnt = counts_sc[...]
        inc = cnt
        shift = 1
        while shift < nb:
            inc = inc + jnp.concatenate(
                [jnp.zeros((1, shift), jnp.float32), inc[:, :-shift]], axis=1)
            shift *= 2
        offs = jnp.concatenate(
            [jnp.zeros((1, 1), jnp.float32), inc[:, :-1]], axis=1)
        offs_sc[...] = offs
        pad = jnp.full((1, nb), float(t), dtype=jnp.float32)
        offs_ref[0, :, :] = jnp.concatenate([offs, pad], axis=1).astype(
            jnp.int32)
        counts_sc[...] = jnp.zeros_like(counts_sc)

    @pl.when(ph == 1)
    def _():
        rr = lax.broadcasted_iota(jnp.int32, (cs, cs), 0)
        cc = lax.broadcasted_iota(jnp.int32, (cs, cs), 1)
        lt = (rr >= cc).astype(jnp.float32)
        csum = jax.lax.dot_general(
            lt, oh, (((1,), (0,)), ((), ())),
            preferred_element_type=jnp.float32)
        inc_global = csum + counts_sc[...]
        rank_f = jnp.sum(oh * (offs_sc[...] + inc_global - 1.0), axis=1)
        rank_ref[0, 0, :] = (rank_f + 0.5).astype(jnp.int32) + h * t
        counts_sc[...] = counts_sc[...] + jnp.sum(oh, axis=0, keepdims=True)


def _ranks(buckets3, nb, cs):
    n_hashes, _, T = buckets3.shape
    nc = T // cs
    rank, offs = pl.pallas_call(
        functools.partial(_rank_kernel, cs=cs, nc=nc, nb=nb, t=T),
        grid=(n_hashes, 2, nc),
        in_specs=[pl.BlockSpec((1, 1, cs), lambda h, ph, c: (h, 0, c))],
        out_specs=[
            pl.BlockSpec((1, 1, cs), lambda h, ph, c: (h, 0, c)),
            pl.BlockSpec((1, 1, 2 * nb), lambda h, ph, c: (h, 0, 0)),
        ],
        out_shape=[
            jax.ShapeDtypeStruct((n_hashes, 1, T), jnp.int32),
            jax.ShapeDtypeStruct((n_hashes, 1, 2 * nb), jnp.int32),
        ],
        scratch_shapes=[
            pltpu.VMEM((1, nb), jnp.float32),
            pltpu.VMEM((1, nb), jnp.float32),
        ],
        compiler_params=pltpu.CompilerParams(
            dimension_semantics=("arbitrary", "arbitrary", "arbitrary")),
    )(buckets3)
    return rank, offs


def _sc_sort_scatter(kv_flat, rank_g):
    T, DK = kv_flat.shape
    NH = rank_g.shape[0]
    info = plsc.get_sparse_core_info()
    nw = info.num_cores * info.num_subcores
    rpw = T // nw
    nch = rpw // 128
    mesh = plsc.VectorSubcoreMesh(core_axis_name="c", subcore_axis_name="s")

    @functools.partial(
        pl.kernel, mesh=mesh,
        out_type=jax.ShapeDtypeStruct((NH * T, DK), jnp.float32),
        scratch_types=[pltpu.VMEM((nch, 128), jnp.int32),
                       pltpu.VMEM((128, DK), jnp.float32),
                       pltpu.SemaphoreType.DMA],
    )
    def sortk(kv_hbm, rank_hbm, kvs_hbm, idx_v, rows, sem):
        wid = lax.axis_index("s") * info.num_cores + lax.axis_index("c")
        base = wid * rpw

        def per_hash(h, carry):
            for j in range(nch):
                pltpu.sync_copy(rank_hbm.at[h, pl.ds(base + j * 128, 128)],
                                idx_v.at[j])
            for j in range(nch):
                pltpu.sync_copy(kv_hbm.at[pl.ds(base + j * 128, 128)], rows)
                pltpu.async_copy(rows, kvs_hbm.at[idx_v.at[j]], sem).wait()
            return carry

        lax.fori_loop(0, NH, per_hash, 0)

    return sortk(kv_flat, rank_g)


def _sc_unsort_gather(out_sorted, rank_g):
    TT, D = out_sorted.shape
    NH, T = rank_g.shape
    info = plsc.get_sparse_core_info()
    nw = info.num_cores * info.num_subcores
    rpw = T // nw
    nch = rpw // 128
    mesh = plsc.VectorSubcoreMesh(core_axis_name="c", subcore_axis_name="s")

    @functools.partial(
        pl.kernel, mesh=mesh,
        out_type=jax.ShapeDtypeStruct((NH * T, D), jnp.float32),
        scratch_types=[pltpu.VMEM((nch, 128), jnp.int32),
                       pltpu.VMEM((128, D), jnp.float32),
                       pltpu.SemaphoreType.DMA],
    )
    def gatherk(os_hbm, rank_hbm, og_hbm, idx_v, rows, sem):
        wid = lax.axis_index("s") * info.num_cores + lax.axis_index("c")
        base = wid * rpw

        def per_hash(h, carry):
            for j in range(nch):
                pltpu.sync_copy(rank_hbm.at[h, pl.ds(base + j * 128, 128)],
                                idx_v.at[j])
            for j in range(nch):
                pltpu.async_copy(os_hbm.at[idx_v.at[j]], rows, sem).wait()
                pltpu.sync_copy(
                    rows, og_hbm.at[pl.ds(h * T + base + j * 128, 128)])
            return carry

        lax.fori_loop(0, NH, per_hash, 0)

    return gatherk(out_sorted, rank_g)


def _attn_kernel(offs_ref, q_ref, kv_ref, o_ref,
                 *, bq, bk, nb, t, d, scale):
    qi = pl.program_id(1)
    off = offs_ref[0, 0, :]
    off32 = off[:nb]
    qlo = qi * bq
    qhi = qlo + bq - 1

    kv_start = jnp.max(jnp.where(off32 <= qlo, off32, 0))
    kv_end = jnp.min(jnp.where(off > qhi, off, t))
    ks_blk = kv_start // bk
    ke_blk = (kv_end + bk - 1) // bk

    p_q = qlo + lax.broadcasted_iota(jnp.int32, (bq, 1), 0)
    bq_id = jnp.sum((off32[None, :] <= p_q).astype(jnp.int32), axis=1) - 1

    q = q_ref[0][:, :d]

    def body(ki, carry):
        m, l, acc = carry
        koff = ki * bk
        kv = kv_ref[0, pl.ds(koff, bk), :]
        k = kv[:, :d]
        v = kv[:, d:]
        s = jax.lax.dot_general(
            q, k, (((1,), (1,)), ((), ())),
            preferred_element_type=jnp.float32) * scale
        p_k = koff + lax.broadcasted_iota(jnp.int32, (bk, 1), 0)
        bk_id = jnp.sum((off32[None, :] <= p_k).astype(jnp.int32), axis=1) - 1
        mask = bq_id[:, None] == bk_id[None, :]
        s = jnp.where(mask, s, -1e30)
        m_new = jnp.maximum(m, jnp.max(s, axis=1, keepdims=True))
        alpha = jnp.exp(m - m_new)
        p = jnp.exp(s - m_new)
        l_new = l * alpha + jnp.sum(p, axis=1, keepdims=True)
        acc_new = acc * alpha + jax.lax.dot_general(
            p, v, (((1,), (0,)), ((), ())),
            preferred_element_type=jnp.float32)
        return m_new, l_new, acc_new

    m0 = jnp.full((bq, 1), -1e30, dtype=jnp.float32)
    l0 = jnp.zeros((bq, 1), dtype=jnp.float32)
    a0 = jnp.zeros((bq, d), dtype=jnp.float32)
    m, l, acc = lax.fori_loop(ks_blk, ke_blk, body, (m0, l0, a0))
    o_ref[0] = jnp.concatenate(
        [acc / l, jnp.zeros((bq, d), dtype=jnp.float32)], axis=1)


def _attention_sorted(kvs3, offs3, bq_block, bk_block):
    n_hashes, T, DK = kvs3.shape
    d = DK // 2
    nb = offs3.shape[2] // 2
    nq = T // bq_block
    scale = 1.0 / math.sqrt(d)
    return pl.pallas_call(
        functools.partial(_attn_kernel, bq=bq_block, bk=bk_block,
                          nb=nb, t=T, d=d, scale=scale),
        grid=(n_hashes, nq),
        in_specs=[
            pl.BlockSpec((1, 1, 2 * nb), lambda h, qi: (h, 0, 0)),
            pl.BlockSpec((1, bq_block, DK), lambda h, qi: (h, qi, 0)),
            pl.BlockSpec((1, T, DK), lambda h, qi: (h, 0, 0)),
        ],
        out_specs=pl.BlockSpec((1, bq_block, DK), lambda h, qi: (h, qi, 0)),
        out_shape=jax.ShapeDtypeStruct((n_hashes, T, DK), jnp.float32),
        compiler_params=pltpu.CompilerParams(
            dimension_semantics=("arbitrary", "arbitrary")),
    )(offs3, kvs3, kvs3)


def _sum_kernel(g_ref, o_ref, *, inv_nh, d):
    o_ref[...] = jnp.sum(g_ref[...], axis=0)[:, :d] * inv_nh


def _sum_hashes(og3, row_block):
    NH, T, DK = og3.shape
    d = DK // 2
    return pl.pallas_call(
        functools.partial(_sum_kernel, inv_nh=1.0 / NH, d=d),
        grid=(T // row_block,),
        in_specs=[pl.BlockSpec((NH, row_block, DK), lambda i: (0, i, 0))],
        out_specs=pl.BlockSpec((row_block, d), lambda i: (i, 0)),
        out_shape=jax.ShapeDtypeStruct((T, d), jnp.float32),
    )(og3)


def _outproj_kernel(y_ref, wo_ref, bo_ref, o_ref):
    o_ref[...] = jax.lax.dot_general(
        y_ref[...], wo_ref[...], (((1,), (1,)), ((), ())),
        preferred_element_type=jnp.float32) + bo_ref[...]


def _outproj(y2, Wo, bo, row_block):
    S, DM = y2.shape
    return pl.pallas_call(
        _outproj_kernel,
        grid=(S // row_block,),
        in_specs=[
            pl.BlockSpec((row_block, DM), lambda i: (i, 0)),
            pl.BlockSpec((DM, DM), lambda i: (0, 0)),
            pl.BlockSpec((1, DM), lambda i: (0, 0)),
        ],
        out_specs=pl.BlockSpec((row_block, DM), lambda i: (i, 0)),
        out_shape=jax.ShapeDtypeStruct((S, DM), jnp.float32),
    )(y2, Wo, bo)


def kernel(x, Wqk, bqk, Wv, bv, Wo, bo, rotations):
    batch, S, DM = x.shape
    n_hashes, H, D, C = rotations.shape
    T = batch * H * S
    nb = 2 * C

    x2 = x.reshape(batch * S, DM)
    row_block = min(256, batch * S)
    qk2, v2 = _project(x2, Wqk, bqk.reshape(1, DM), Wv, bv.reshape(1, DM),
                       row_block)

    qk_flat = qk2.reshape(batch, S, H, D).transpose(0, 2, 1, 3).reshape(T, D)
    v_flat = v2.reshape(batch, S, H, D).transpose(0, 2, 1, 3).reshape(T, D)

    rot_flat = rotations.transpose(1, 2, 0, 3).reshape(H, D, n_hashes * C)
    bkt = _hash_buckets(qk_flat, rot_flat, n_hashes, C)
    buckets3 = bkt.transpose(1, 0, 2).reshape(n_hashes, 1, T)

    rank3, offs3 = _ranks(buckets3, nb, min(512, T))
    rank_g = rank3.reshape(n_hashes, T)

    kv_flat = jnp.concatenate([qk_flat, v_flat], axis=1)
    kvs = _sc_sort_scatter(kv_flat, rank_g)
    kvs3 = kvs.reshape(n_hashes, T, 2 * D)

    bq_block = min(256, T)
    bk_block = min(256, T)
    os3 = _attention_sorted(kvs3, offs3, bq_block, bk_block)

    og = _sc_unsort_gather(os3.reshape(n_hashes * T, 2 * D), rank_g)
    out_flat = _sum_hashes(og.reshape(n_hashes, T, 2 * D), min(1024, T))

    y2 = out_flat.reshape(batch, H, S, D).transpose(0, 2, 1, 3).reshape(
        batch * S, DM)
    out = _outproj(y2, Wo, bo.reshape(1, DM), row_block)
    return out.reshape(batch, S, DM)


# speedup vs baseline: 5.1557x; 1.0988x over previous
import functools
import math

import jax
import jax.numpy as jnp
from jax import lax
from jax.experimental import pallas as pl
from jax.experimental.pallas import tpu as pltpu
from jax.experimental.pallas import tpu_sc as plsc


def _proj_kernel(x_ref, wqk_ref, bqk_ref, wv_ref, bv_ref, qk_ref, v_ref):
    x = x_ref[...]
    qk_ref[...] = jax.lax.dot_general(
        x, wqk_ref[...], (((1,), (1,)), ((), ())),
        preferred_element_type=jnp.float32) + bqk_ref[...]
    v_ref[...] = jax.lax.dot_general(
        x, wv_ref[...], (((1,), (1,)), ((), ())),
        preferred_element_type=jnp.float32) + bv_ref[...]


def _project(x2, Wqk, bqk, Wv, bv, row_block):
    S, DM = x2.shape
    return pl.pallas_call(
        _proj_kernel,
        grid=(S // row_block,),
        in_specs=[
            pl.BlockSpec((row_block, DM), lambda i: (i, 0)),
            pl.BlockSpec((DM, DM), lambda i: (0, 0)),
            pl.BlockSpec((1, DM), lambda i: (0, 0)),
            pl.BlockSpec((DM, DM), lambda i: (0, 0)),
            pl.BlockSpec((1, DM), lambda i: (0, 0)),
        ],
        out_specs=[
            pl.BlockSpec((row_block, DM), lambda i: (i, 0)),
            pl.BlockSpec((row_block, DM), lambda i: (i, 0)),
        ],
        out_shape=[
            jax.ShapeDtypeStruct((S, DM), jnp.float32),
            jax.ShapeDtypeStruct((S, DM), jnp.float32),
        ],
    )(x2, Wqk, bqk, Wv, bv)


def _hash_kernel(qk_ref, rot_ref, bkt_ref, offs_ref, counts_sc,
                 *, n_hashes, rot_size, n_heads, t):
    hh = pl.program_id(0)
    nb = 2 * rot_size
    q = qk_ref[...]
    r = rot_ref[0]
    rot = jax.lax.dot_general(
        q, r, (((1,), (0,)), ((), ())), preferred_element_type=jnp.float32)
    s = rot.shape[0]

    @pl.when(hh == 0)
    def _():
        counts_sc[...] = jnp.zeros_like(counts_sc)

    lanes = lax.broadcasted_iota(jnp.int32, (s, nb), 1)
    for h in range(n_hashes):
        sub = rot[:, h * rot_size:(h + 1) * rot_size]
        full = jnp.concatenate([sub, -sub], axis=1)
        b = jnp.argmax(full, axis=1).astype(jnp.int32)
        bkt_ref[0, h, :] = b
        oh = (b[:, None] == lanes).astype(jnp.float32)
        counts_sc[h, :] = counts_sc[h, :] + jnp.sum(oh, axis=0)

    @pl.when(hh == n_heads - 1)
    def _():
        cnt = counts_sc[...]
        inc = cnt
        shift = 1
        while shift < nb:
            inc = inc + jnp.concatenate(
                [jnp.zeros((n_hashes, shift), jnp.float32),
                 inc[:, :-shift]], axis=1)
            shift *= 2
        offs = jnp.concatenate(
            [jnp.zeros((n_hashes, 1), jnp.float32), inc[:, :-1]], axis=1)
        pad = jnp.full((n_hashes, nb), float(t), dtype=jnp.float32)
        offs_ref[...] = jnp.concatenate(
            [offs, pad], axis=1).astype(jnp.int32)[:, None, :]


def _hash_buckets(qk_heads, rot_flat, n_hashes, rot_size):
    H = rot_flat.shape[0]
    D = rot_flat.shape[1]
    S = qk_heads.shape[0] // H
    nb = 2 * rot_size
    return pl.pallas_call(
        functools.partial(_hash_kernel, n_hashes=n_hashes, rot_size=rot_size,
                          n_heads=H, t=H * S),
        grid=(H,),
        in_specs=[
            pl.BlockSpec((S, D), lambda h: (h, 0)),
            pl.BlockSpec((1, D, n_hashes * rot_size), lambda h: (h, 0, 0)),
        ],
        out_specs=[
            pl.BlockSpec((1, n_hashes, S), lambda h: (h, 0, 0)),
            pl.BlockSpec((n_hashes, 1, 2 * nb), lambda h: (0, 0, 0)),
        ],
        out_shape=[
            jax.ShapeDtypeStruct((H, n_hashes, S), jnp.int32),
            jax.ShapeDtypeStruct((n_hashes, 1, 2 * nb), jnp.int32),
        ],
        scratch_shapes=[pltpu.VMEM((n_hashes, nb), jnp.float32)],
    )(qk_heads, rot_flat)


def _rank_kernel(tri_ref, bkt_ref, offs_ref, rank_ref, counts_sc,
                 *, cs, nb, t):
    h = pl.program_id(0)
    c = pl.program_id(1)

    b = bkt_ref[0, 0, :]
    lanes = lax.broadcasted_iota(jnp.int32, (cs, nb), 1)
    oh = (b[:, None] == lanes).astype(jnp.float32)

    @pl.when(c == 0)
    def _():
        counts_sc[...] = jnp.zeros_like(counts_sc)

    csum = jax.lax.dot_general(
        tri_ref[...], oh.astype(jnp.bfloat16), (((1,), (0,)), ((), ())),
        preferred_element_type=jnp.float32)
    offs = offs_ref[0, 0, :nb].astype(jnp.float32)[None, :]
    inc_global = csum + counts_sc[...]
    rank_f = jnp.sum(oh * (offs + inc_global - 1.0), axis=1)
    rank_ref[0, 0, :] = (rank_f + 0.5).astype(jnp.int32) + h * t
    counts_sc[...] = counts_sc[...] + jnp.sum(oh, axis=0, keepdims=True)


def _ranks(buckets3, offs3, nb, cs):
    n_hashes, _, T = buckets3.shape
    nc = T // cs
    rr = lax.broadcasted_iota(jnp.int32, (cs, cs), 0)
    cc = lax.broadcasted_iota(jnp.int32, (cs, cs), 1)
    tri = (rr >= cc).astype(jnp.bfloat16)
    return pl.pallas_call(
        functools.partial(_rank_kernel, cs=cs, nb=nb, t=T),
        grid=(n_hashes, nc),
        in_specs=[
            pl.BlockSpec((cs, cs), lambda h, c: (0, 0)),
            pl.BlockSpec((1, 1, cs), lambda h, c: (h, 0, c)),
            pl.BlockSpec((1, 1, 2 * nb), lambda h, c: (h, 0, 0)),
        ],
        out_specs=pl.BlockSpec((1, 1, cs), lambda h, c: (h, 0, c)),
        out_shape=jax.ShapeDtypeStruct((n_hashes, 1, T), jnp.int32),
        scratch_shapes=[pltpu.VMEM((1, nb), jnp.float32)],
        compiler_params=pltpu.CompilerParams(
            dimension_semantics=("parallel", "arbitrary")),
    )(tri, buckets3, offs3)


def _sc_sort_scatter(kv_flat, rank_g):
    T, DK = kv_flat.shape
    NH = rank_g.shape[0]
    info = plsc.get_sparse_core_info()
    nw = info.num_cores * info.num_subcores
    rpw = T // nw
    nch = rpw // 128
    mesh = plsc.VectorSubcoreMesh(core_axis_name="c", subcore_axis_name="s")

    @functools.partial(
        pl.kernel, mesh=mesh,
        out_type=jax.ShapeDtypeStruct((NH * T, DK), jnp.float32),
        scratch_types=[pltpu.VMEM((nch, 128), jnp.int32),
                       pltpu.VMEM((128, DK), jnp.float32),
                       pltpu.SemaphoreType.DMA],
    )
    def sortk(kv_hbm, rank_hbm, kvs_hbm, idx_v, rows, sem):
        wid = lax.axis_index("s") * info.num_cores + lax.axis_index("c")
        base = wid * rpw

        def per_hash(h, carry):
            for j in range(nch):
                pltpu.sync_copy(rank_hbm.at[h, pl.ds(base + j * 128, 128)],
                                idx_v.at[j])
            for j in range(nch):
                pltpu.sync_copy(kv_hbm.at[pl.ds(base + j * 128, 128)], rows)
                pltpu.async_copy(rows, kvs_hbm.at[idx_v.at[j]], sem).wait()
            return carry

        lax.fori_loop(0, NH, per_hash, 0)

    return sortk(kv_flat, rank_g)


def _sc_unsort_gather(out_sorted, rank_g):
    TT, D = out_sorted.shape
    NH, T = rank_g.shape
    info = plsc.get_sparse_core_info()
    nw = info.num_cores * info.num_subcores
    rpw = T // nw
    nch = rpw // 128
    mesh = plsc.VectorSubcoreMesh(core_axis_name="c", subcore_axis_name="s")

    @functools.partial(
        pl.kernel, mesh=mesh,
        out_type=jax.ShapeDtypeStruct((NH * T, D), jnp.float32),
        scratch_types=[pltpu.VMEM((nch, 128), jnp.int32),
                       pltpu.VMEM((128, D), jnp.float32),
                       pltpu.SemaphoreType.DMA],
    )
    def gatherk(os_hbm, rank_hbm, og_hbm, idx_v, rows, sem):
        wid = lax.axis_index("s") * info.num_cores + lax.axis_index("c")
        base = wid * rpw

        def per_hash(h, carry):
            for j in range(nch):
                pltpu.sync_copy(rank_hbm.at[h, pl.ds(base + j * 128, 128)],
                                idx_v.at[j])
            for j in range(nch):
                pltpu.async_copy(os_hbm.at[idx_v.at[j]], rows, sem).wait()
                pltpu.sync_copy(
                    rows, og_hbm.at[pl.ds(h * T + base + j * 128, 128)])
            return carry

        lax.fori_loop(0, NH, per_hash, 0)

    return gatherk(out_sorted, rank_g)


def _attn_kernel(offs_ref, q_ref, kv_ref, o_ref,
                 *, bq, bk, nb, t, d, scale):
    qi = pl.program_id(1)
    off = offs_ref[0, 0, :]
    off32 = off[:nb]
    qlo = qi * bq
    qhi = qlo + bq - 1

    kv_start = jnp.max(jnp.where(off32 <= qlo, off32, 0))
    kv_end = jnp.min(jnp.where(off > qhi, off, t))
    ks_blk = kv_start // bk
    ke_blk = (kv_end + bk - 1) // bk

    p_q = qlo + lax.broadcasted_iota(jnp.int32, (bq, 1), 0)
    bq_id = jnp.sum((off32[None, :] <= p_q).astype(jnp.int32), axis=1) - 1

    q = q_ref[0][:, :d].astype(jnp.bfloat16)

    def body(ki, carry):
        m, l, acc = carry
        koff = ki * bk
        kv = kv_ref[0, pl.ds(koff, bk), :]
        k = kv[:, :d].astype(jnp.bfloat16)
        v = kv[:, d:].astype(jnp.bfloat16)
        s = jax.lax.dot_general(
            q, k, (((1,), (1,)), ((), ())),
            preferred_element_type=jnp.float32) * scale
        p_k = koff + lax.broadcasted_iota(jnp.int32, (bk, 1), 0)
        bk_id = jnp.sum((off32[None, :] <= p_k).astype(jnp.int32), axis=1) - 1
        mask = bq_id[:, None] == bk_id[None, :]
        s = jnp.where(mask, s, -1e30)
        m_new = jnp.maximum(m, jnp.max(s, axis=1, keepdims=True))
        alpha = jnp.exp(m - m_new)
        p = jnp.exp(s - m_new)
        l_new = l * alpha + jnp.sum(p, axis=1, keepdims=True)
        acc_new = acc * alpha + jax.lax.dot_general(
            p.astype(jnp.bfloat16), v, (((1,), (0,)), ((), ())),
            preferred_element_type=jnp.float32)
        return m_new, l_new, acc_new

    m0 = jnp.full((bq, 1), -1e30, dtype=jnp.float32)
    l0 = jnp.zeros((bq, 1), dtype=jnp.float32)
    a0 = jnp.zeros((bq, d), dtype=jnp.float32)
    m, l, acc = lax.fori_loop(ks_blk, ke_blk, body, (m0, l0, a0))
    o_ref[0, :, :d] = acc / l


def _attention_sorted(kvs3, offs3, bq_block, bk_block):
    n_hashes, T, DK = kvs3.shape
    d = DK // 2
    nb = offs3.shape[2] // 2
    nq = T // bq_block
    scale = 1.0 / math.sqrt(d)
    return pl.pallas_call(
        functools.partial(_attn_kernel, bq=bq_block, bk=bk_block,
                          nb=nb, t=T, d=d, scale=scale),
        grid=(n_hashes, nq),
        in_specs=[
            pl.BlockSpec((1, 1, 2 * nb), lambda h, qi: (h, 0, 0)),
            pl.BlockSpec((1, bq_block, DK), lambda h, qi: (h, qi, 0)),
            pl.BlockSpec((1, T, DK), lambda h, qi: (h, 0, 0)),
        ],
        out_specs=pl.BlockSpec((1, bq_block, DK), lambda h, qi: (h, qi, 0)),
        out_shape=jax.ShapeDtypeStruct((n_hashes, T, DK), jnp.float32),
        compiler_params=pltpu.CompilerParams(
            dimension_semantics=("arbitrary", "parallel")),
    )(offs3, kvs3, kvs3)


def _sum_kernel(g_ref, o_ref, *, inv_nh, d):
    o_ref[...] = jnp.sum(g_ref[...][:, :, :d], axis=0) * inv_nh


def _sum_hashes(og3, row_block):
    NH, T, DK = og3.shape
    d = DK // 2
    return pl.pallas_call(
        functools.partial(_sum_kernel, inv_nh=1.0 / NH, d=d),
        grid=(T // row_block,),
        in_specs=[pl.BlockSpec((NH, row_block, DK), lambda i: (0, i, 0))],
        out_specs=pl.BlockSpec((row_block, d), lambda i: (i, 0)),
        out_shape=jax.ShapeDtypeStruct((T, d), jnp.float32),
    )(og3)


def _outproj_kernel(y_ref, wo_ref, bo_ref, o_ref):
    o_ref[...] = jax.lax.dot_general(
        y_ref[...], wo_ref[...], (((1,), (1,)), ((), ())),
        preferred_element_type=jnp.float32) + bo_ref[...]


def _outproj(y2, Wo, bo, row_block):
    S, DM = y2.shape
    return pl.pallas_call(
        _outproj_kernel,
        grid=(S // row_block,),
        in_specs=[
            pl.BlockSpec((row_block, DM), lambda i: (i, 0)),
            pl.BlockSpec((DM, DM), lambda i: (0, 0)),
            pl.BlockSpec((1, DM), lambda i: (0, 0)),
        ],
        out_specs=pl.BlockSpec((row_block, DM), lambda i: (i, 0)),
        out_shape=jax.ShapeDtypeStruct((S, DM), jnp.float32),
    )(y2, Wo, bo)


def kernel(x, Wqk, bqk, Wv, bv, Wo, bo, rotations):
    batch, S, DM = x.shape
    n_hashes, H, D, C = rotations.shape
    T = batch * H * S
    nb = 2 * C

    x2 = x.reshape(batch * S, DM)
    row_block = min(256, batch * S)
    qk2, v2 = _project(x2, Wqk, bqk.reshape(1, DM), Wv, bv.reshape(1, DM),
                       row_block)

    qk_flat = qk2.reshape(batch, S, H, D).transpose(0, 2, 1, 3).reshape(T, D)
    v_flat = v2.reshape(batch, S, H, D).transpose(0, 2, 1, 3).reshape(T, D)

    rot_flat = rotations.transpose(1, 2, 0, 3).reshape(H, D, n_hashes * C)
    bkt, offs3 = _hash_buckets(qk_flat, rot_flat, n_hashes, C)
    buckets3 = bkt.transpose(1, 0, 2).reshape(n_hashes, 1, T)

    rank3 = _ranks(buckets3, offs3, nb, min(512, T))
    rank_g = rank3.reshape(n_hashes, T)

    kv_flat = jnp.concatenate([qk_flat, v_flat], axis=1)
    kvs = _sc_sort_scatter(kv_flat, rank_g)
    kvs3 = kvs.reshape(n_hashes, T, 2 * D)

    bq_block = min(256, T)
    bk_block = min(512, T)
    os3 = _attention_sorted(kvs3, offs3, bq_block, bk_block)

    og = _sc_unsort_gather(os3.reshape(n_hashes * T, 2 * D), rank_g)
    out_flat = _sum_hashes(og.reshape(n_hashes, T, 2 * D), min(1024, T))

    y2 = out_flat.reshape(batch, H, S, D).transpose(0, 2, 1, 3).reshape(
        batch * S, DM)
    out = _outproj(y2, Wo, bo.reshape(1, DM), row_block)
    return out.reshape(batch, S, DM)


# speedup vs baseline: 5.5309x; 1.0728x over previous
import functools
import math

import jax
import jax.numpy as jnp
from jax import lax
from jax.experimental import pallas as pl
from jax.experimental.pallas import tpu as pltpu
from jax.experimental.pallas import tpu_sc as plsc


def _proj_kernel(x_ref, wqk_ref, bqk_ref, wv_ref, bv_ref, qk_ref, v_ref):
    x = x_ref[...]
    qk_ref[...] = jax.lax.dot_general(
        x, wqk_ref[...], (((1,), (1,)), ((), ())),
        preferred_element_type=jnp.float32) + bqk_ref[...]
    v_ref[...] = jax.lax.dot_general(
        x, wv_ref[...], (((1,), (1,)), ((), ())),
        preferred_element_type=jnp.float32) + bv_ref[...]


def _project(x2, Wqk, bqk, Wv, bv, row_block):
    S, DM = x2.shape
    return pl.pallas_call(
        _proj_kernel,
        grid=(S // row_block,),
        in_specs=[
            pl.BlockSpec((row_block, DM), lambda i: (i, 0)),
            pl.BlockSpec((DM, DM), lambda i: (0, 0)),
            pl.BlockSpec((1, DM), lambda i: (0, 0)),
            pl.BlockSpec((DM, DM), lambda i: (0, 0)),
            pl.BlockSpec((1, DM), lambda i: (0, 0)),
        ],
        out_specs=[
            pl.BlockSpec((row_block, DM), lambda i: (i, 0)),
            pl.BlockSpec((row_block, DM), lambda i: (i, 0)),
        ],
        out_shape=[
            jax.ShapeDtypeStruct((S, DM), jnp.float32),
            jax.ShapeDtypeStruct((S, DM), jnp.float32),
        ],
    )(x2, Wqk, bqk, Wv, bv)


def _hash_kernel(qk_ref, rot_ref, bkt_ref, offs_ref, counts_sc,
                 *, n_hashes, rot_size, n_heads, t):
    hh = pl.program_id(0)
    nb = 2 * rot_size
    q = qk_ref[...]
    r = rot_ref[0]
    rot = jax.lax.dot_general(
        q, r, (((1,), (0,)), ((), ())), preferred_element_type=jnp.float32)
    s = rot.shape[0]

    @pl.when(hh == 0)
    def _():
        counts_sc[...] = jnp.zeros_like(counts_sc)

    lanes = lax.broadcasted_iota(jnp.int32, (s, nb), 1)
    for h in range(n_hashes):
        sub = rot[:, h * rot_size:(h + 1) * rot_size]
        full = jnp.concatenate([sub, -sub], axis=1)
        b = jnp.argmax(full, axis=1).astype(jnp.int32)
        bkt_ref[0, h, :] = b
        oh = (b[:, None] == lanes).astype(jnp.float32)
        counts_sc[h, :] = counts_sc[h, :] + jnp.sum(oh, axis=0)

    @pl.when(hh == n_heads - 1)
    def _():
        cnt = counts_sc[...]
        inc = cnt
        shift = 1
        while shift < nb:
            inc = inc + jnp.concatenate(
                [jnp.zeros((n_hashes, shift), jnp.float32),
                 inc[:, :-shift]], axis=1)
            shift *= 2
        offs = jnp.concatenate(
            [jnp.zeros((n_hashes, 1), jnp.float32), inc[:, :-1]], axis=1)
        pad = jnp.full((n_hashes, nb), float(t), dtype=jnp.float32)
        offs_ref[...] = jnp.concatenate(
            [offs, pad], axis=1).astype(jnp.int32)[:, None, :]


def _hash_buckets(qk_heads, rot_flat, n_hashes, rot_size):
    H = rot_flat.shape[0]
    D = rot_flat.shape[1]
    S = qk_heads.shape[0] // H
    nb = 2 * rot_size
    return pl.pallas_call(
        functools.partial(_hash_kernel, n_hashes=n_hashes, rot_size=rot_size,
                          n_heads=H, t=H * S),
        grid=(H,),
        in_specs=[
            pl.BlockSpec((S, D), lambda h: (h, 0)),
            pl.BlockSpec((1, D, n_hashes * rot_size), lambda h: (h, 0, 0)),
        ],
        out_specs=[
            pl.BlockSpec((1, n_hashes, S), lambda h: (h, 0, 0)),
            pl.BlockSpec((n_hashes, 1, 2 * nb), lambda h: (0, 0, 0)),
        ],
        out_shape=[
            jax.ShapeDtypeStruct((H, n_hashes, S), jnp.int32),
            jax.ShapeDtypeStruct((n_hashes, 1, 2 * nb), jnp.int32),
        ],
        scratch_shapes=[pltpu.VMEM((n_hashes, nb), jnp.float32)],
    )(qk_heads, rot_flat)


def _rank_kernel(tri_ref, bkt_ref, offs_ref, rank_ref, counts_sc,
                 *, cs, nb, t):
    h = pl.program_id(0)
    c = pl.program_id(1)

    b = bkt_ref[0, 0, :]
    lanes = lax.broadcasted_iota(jnp.int32, (cs, nb), 1)
    oh = (b[:, None] == lanes).astype(jnp.float32)

    @pl.when(c == 0)
    def _():
        counts_sc[...] = jnp.zeros_like(counts_sc)

    csum = jax.lax.dot_general(
        tri_ref[...], oh.astype(jnp.bfloat16), (((1,), (0,)), ((), ())),
        preferred_element_type=jnp.float32)
    offs = offs_ref[0, 0, :nb].astype(jnp.float32)[None, :]
    inc_global = csum + counts_sc[...]
    rank_f = jnp.sum(oh * (offs + inc_global - 1.0), axis=1)
    rank_ref[0, 0, :] = (rank_f + 0.5).astype(jnp.int32) + h * t
    counts_sc[...] = counts_sc[...] + jnp.sum(oh, axis=0, keepdims=True)


def _ranks(buckets3, offs3, nb, cs):
    n_hashes, _, T = buckets3.shape
    nc = T // cs
    rr = lax.broadcasted_iota(jnp.int32, (cs, cs), 0)
    cc = lax.broadcasted_iota(jnp.int32, (cs, cs), 1)
    tri = (rr >= cc).astype(jnp.bfloat16)
    return pl.pallas_call(
        functools.partial(_rank_kernel, cs=cs, nb=nb, t=T),
        grid=(n_hashes, nc),
        in_specs=[
            pl.BlockSpec((cs, cs), lambda h, c: (0, 0)),
            pl.BlockSpec((1, 1, cs), lambda h, c: (h, 0, c)),
            pl.BlockSpec((1, 1, 2 * nb), lambda h, c: (h, 0, 0)),
        ],
        out_specs=pl.BlockSpec((1, 1, cs), lambda h, c: (h, 0, c)),
        out_shape=jax.ShapeDtypeStruct((n_hashes, 1, T), jnp.int32),
        scratch_shapes=[pltpu.VMEM((1, nb), jnp.float32)],
        compiler_params=pltpu.CompilerParams(
            dimension_semantics=("parallel", "arbitrary")),
    )(tri, buckets3, offs3)


def _sc_sort_scatter(kv_flat, rank_g):
    T, DK = kv_flat.shape
    NH = rank_g.shape[0]
    info = plsc.get_sparse_core_info()
    nw = info.num_cores * info.num_subcores
    rpw = T // nw
    nch = rpw // 128
    mesh = plsc.VectorSubcoreMesh(core_axis_name="c", subcore_axis_name="s")

    @functools.partial(
        pl.kernel, mesh=mesh,
        out_type=jax.ShapeDtypeStruct((NH * T, DK), jnp.float32),
        scratch_types=[pltpu.VMEM((nch, 128), jnp.int32),
                       pltpu.VMEM((rpw, DK), jnp.float32),
                       pltpu.SemaphoreType.DMA,
                       pltpu.SemaphoreType.DMA],
    )
    def sortk(kv_hbm, rank_hbm, kvs_hbm, idx_v, rows, semi, semw):
        wid = lax.axis_index("s") * info.num_cores + lax.axis_index("c")
        base = wid * rpw
        pltpu.sync_copy(kv_hbm.at[pl.ds(base, rpw)], rows)

        def per_hash(h, carry):
            loads = [
                pltpu.async_copy(rank_hbm.at[h, pl.ds(base + j * 128, 128)],
                                 idx_v.at[j], semi)
                for j in range(nch)
            ]
            for hd in loads:
                hd.wait()
            stores = [
                pltpu.async_copy(rows.at[pl.ds(j * 128, 128)],
                                 kvs_hbm.at[idx_v.at[j]], semw)
                for j in range(nch)
            ]
            for hd in stores:
                hd.wait()
            return carry

        lax.fori_loop(0, NH, per_hash, 0)

    return sortk(kv_flat, rank_g)


def _sc_unsort_gather(out_sorted, rank_g):
    TT, D = out_sorted.shape
    NH, T = rank_g.shape
    info = plsc.get_sparse_core_info()
    nw = info.num_cores * info.num_subcores
    rpw = T // nw
    nch = rpw // 128
    mesh = plsc.VectorSubcoreMesh(core_axis_name="c", subcore_axis_name="s")

    @functools.partial(
        pl.kernel, mesh=mesh,
        out_type=jax.ShapeDtypeStruct((NH * T, D), jnp.float32),
        scratch_types=[pltpu.VMEM((nch, 128), jnp.int32),
                       pltpu.VMEM((rpw, D), jnp.float32),
                       pltpu.SemaphoreType.DMA,
                       pltpu.SemaphoreType.DMA],
    )
    def gatherk(os_hbm, rank_hbm, og_hbm, idx_v, rows, semi, semr):
        wid = lax.axis_index("s") * info.num_cores + lax.axis_index("c")
        base = wid * rpw

        def per_hash(h, carry):
            loads = [
                pltpu.async_copy(rank_hbm.at[h, pl.ds(base + j * 128, 128)],
                                 idx_v.at[j], semi)
                for j in range(nch)
            ]
            for hd in loads:
                hd.wait()
            reads = [
                pltpu.async_copy(os_hbm.at[idx_v.at[j]],
                                 rows.at[pl.ds(j * 128, 128)], semr)
                for j in range(nch)
            ]
            for hd in reads:
                hd.wait()
            pltpu.sync_copy(rows, og_hbm.at[pl.ds(h * T + base, rpw)])
            return carry

        lax.fori_loop(0, NH, per_hash, 0)

    return gatherk(out_sorted, rank_g)


def _attn_kernel(offs_ref, q_ref, kv_ref, o_ref,
                 *, bq, bk, nb, t, d, scale):
    qi = pl.program_id(1)
    off = offs_ref[0, 0, :]
    off32 = off[:nb]
    qlo = qi * bq
    qhi = qlo + bq - 1

    kv_start = jnp.max(jnp.where(off32 <= qlo, off32, 0))
    kv_end = jnp.min(jnp.where(off > qhi, off, t))
    ks_blk = kv_start // bk
    ke_blk = (kv_end + bk - 1) // bk

    p_q = qlo + lax.broadcasted_iota(jnp.int32, (bq, 1), 0)
    bq_id = jnp.sum((off32[None, :] <= p_q).astype(jnp.int32), axis=1) - 1

    q = (q_ref[0][:, :d] * scale).astype(jnp.bfloat16)
    aug_lane = lax.broadcasted_iota(jnp.int32, (bk, 2 * d), 1)

    def body(ki, carry):
        m, acc = carry
        koff = ki * bk
        kv = kv_ref[0, pl.ds(koff, bk), :]
        k = kv[:, :d].astype(jnp.bfloat16)
        vaug = jnp.where(
            aug_lane < d, jnp.roll(kv, -d, axis=1),
            jnp.where(aug_lane == d, 1.0, 0.0)).astype(jnp.bfloat16)
        s = jax.lax.dot_general(
            q, k, (((1,), (1,)), ((), ())),
            preferred_element_type=jnp.float32)
        p_k = koff + lax.broadcasted_iota(jnp.int32, (bk, 1), 0)
        bk_id = jnp.sum((off32[None, :] <= p_k).astype(jnp.int32), axis=1) - 1
        mask = bq_id[:, None] == bk_id[None, :]
        s = jnp.where(mask, s, -1e30)
        m_new = jnp.maximum(m, jnp.max(s, axis=1, keepdims=True))
        alpha = jnp.exp(m - m_new)
        p = jnp.exp(s - m_new)
        acc_new = acc * alpha + jax.lax.dot_general(
            p.astype(jnp.bfloat16), vaug, (((1,), (0,)), ((), ())),
            preferred_element_type=jnp.float32)
        return m_new, acc_new

    m0 = jnp.full((bq, 1), -1e30, dtype=jnp.float32)
    a0 = jnp.zeros((bq, 2 * d), dtype=jnp.float32)
    m, acc = lax.fori_loop(ks_blk, ke_blk, body, (m0, a0))
    o_ref[0, :, :d] = acc[:, :d] / acc[:, d:d + 1]


def _attention_sorted(kvs3, offs3, bq_block, bk_block):
    n_hashes, T, DK = kvs3.shape
    d = DK // 2
    nb = offs3.shape[2] // 2
    nq = T // bq_block
    scale = 1.0 / math.sqrt(d)
    return pl.pallas_call(
        functools.partial(_attn_kernel, bq=bq_block, bk=bk_block,
                          nb=nb, t=T, d=d, scale=scale),
        grid=(n_hashes, nq),
        in_specs=[
            pl.BlockSpec((1, 1, 2 * nb), lambda h, qi: (h, 0, 0)),
            pl.BlockSpec((1, bq_block, DK), lambda h, qi: (h, qi, 0)),
            pl.BlockSpec((1, T, DK), lambda h, qi: (h, 0, 0)),
        ],
        out_specs=pl.BlockSpec((1, bq_block, DK), lambda h, qi: (h, qi, 0)),
        out_shape=jax.ShapeDtypeStruct((n_hashes, T, DK), jnp.float32),
        compiler_params=pltpu.CompilerParams(
            dimension_semantics=("arbitrary", "parallel")),
    )(offs3, kvs3, kvs3)


def _sum_kernel(g_ref, o_ref, *, inv_nh, d):
    o_ref[...] = jnp.sum(g_ref[...][:, :, :d], axis=0) * inv_nh


def _sum_hashes(og3, row_block):
    NH, T, DK = og3.shape
    d = DK // 2
    return pl.pallas_call(
        functools.partial(_sum_kernel, inv_nh=1.0 / NH, d=d),
        grid=(T // row_block,),
        in_specs=[pl.BlockSpec((NH, row_block, DK), lambda i: (0, i, 0))],
        out_specs=pl.BlockSpec((row_block, d), lambda i: (i, 0)),
        out_shape=jax.ShapeDtypeStruct((T, d), jnp.float32),
    )(og3)


def _outproj_kernel(y_ref, wo_ref, bo_ref, o_ref):
    o_ref[...] = jax.lax.dot_general(
        y_ref[...], wo_ref[...], (((1,), (1,)), ((), ())),
        preferred_element_type=jnp.float32) + bo_ref[...]


def _outproj(y2, Wo, bo, row_block):
    S, DM = y2.shape
    return pl.pallas_call(
        _outproj_kernel,
        grid=(S // row_block,),
        in_specs=[
            pl.BlockSpec((row_block, DM), lambda i: (i, 0)),
            pl.BlockSpec((DM, DM), lambda i: (0, 0)),
            pl.BlockSpec((1, DM), lambda i: (0, 0)),
        ],
        out_specs=pl.BlockSpec((row_block, DM), lambda i: (i, 0)),
        out_shape=jax.ShapeDtypeStruct((S, DM), jnp.float32),
    )(y2, Wo, bo)


def kernel(x, Wqk, bqk, Wv, bv, Wo, bo, rotations):
    batch, S, DM = x.shape
    n_hashes, H, D, C = rotations.shape
    T = batch * H * S
    nb = 2 * C

    x2 = x.reshape(batch * S, DM)
    row_block = min(256, batch * S)
    qk2, v2 = _project(x2, Wqk, bqk.reshape(1, DM), Wv, bv.reshape(1, DM),
                       row_block)

    qk_flat = qk2.reshape(batch, S, H, D).transpose(0, 2, 1, 3).reshape(T, D)
    v_flat = v2.reshape(batch, S, H, D).transpose(0, 2, 1, 3).reshape(T, D)

    rot_flat = rotations.transpose(1, 2, 0, 3).reshape(H, D, n_hashes * C)
    bkt, offs3 = _hash_buckets(qk_flat, rot_flat, n_hashes, C)
    buckets3 = bkt.transpose(1, 0, 2).reshape(n_hashes, 1, T)

    rank3 = _ranks(buckets3, offs3, nb, min(512, T))
    rank_g = rank3.reshape(n_hashes, T)

    kv_flat = jnp.concatenate([qk_flat, v_flat], axis=1)
    kvs = _sc_sort_scatter(kv_flat, rank_g)
    kvs3 = kvs.reshape(n_hashes, T, 2 * D)

    bq_block = min(256, T)
    bk_block = min(512, T)
    os3 = _attention_sorted(kvs3, offs3, bq_block, bk_block)

    og = _sc_unsort_gather(os3.reshape(n_hashes * T, 2 * D), rank_g)
    out_flat = _sum_hashes(og.reshape(n_hashes, T, 2 * D), min(1024, T))

    y2 = out_flat.reshape(batch, H, S, D).transpose(0, 2, 1, 3).reshape(
        batch * S, DM)
    out = _outproj(y2, Wo, bo.reshape(1, DM), row_block)
    return out.reshape(batch, S, DM)
